# baseline (speedup 1.0000x reference)
import functools

import jax
import jax.numpy as jnp
from jax import lax
from jax.experimental import pallas as pl
from jax.experimental.pallas import tpu as pltpu
from jax.experimental.pallas import tpu_sc as plsc

_BATCH = 4
_SEQ = 8192
_DIM = 768
_NC = 2
_NS = 16
_NW = _NC * _NS
_LANES = 16

_POS_PER_W = _SEQ // _NW
_ROWS = 32
_CHUNKS = _POS_PER_W // _ROWS
_CHUNK_ELEMS = _ROWS * _DIM


def _sc_add_kernel(x_hbm, enc_hbm, out_hbm, be, bx, sem):
    wid = lax.axis_index("s") * _NC + lax.axis_index("c")
    pos_base = wid * _POS_PER_W

    for chunk in range(_CHUNKS):
        pos0 = pos_base + chunk * _ROWS
        ebase = pos0 * _DIM
        pltpu.sync_copy(enc_hbm.at[pl.ds(ebase, _CHUNK_ELEMS)], be)
        for b in range(_BATCH):
            xbase = (b * _SEQ + pos0) * _DIM
            pltpu.sync_copy(x_hbm.at[pl.ds(xbase, _CHUNK_ELEMS)], bx)

            def body(k, _):
                sl = pl.ds(k * _LANES, _LANES)
                bx[sl] = bx[sl] + be[sl]
                return 0

            lax.fori_loop(0, _CHUNK_ELEMS // _LANES, body, 0)
            pltpu.sync_copy(bx, out_hbm.at[pl.ds(xbase, _CHUNK_ELEMS)])


@jax.jit
def _sc_add(x_flat, enc_flat):
    mesh = plsc.VectorSubcoreMesh(core_axis_name="c", subcore_axis_name="s")
    run = functools.partial(
        pl.kernel,
        mesh=mesh,
        out_type=jax.ShapeDtypeStruct((_BATCH * _SEQ * _DIM,), jnp.float32),
        scratch_types=[
            pltpu.VMEM((_CHUNK_ELEMS,), jnp.float32),
            pltpu.VMEM((_CHUNK_ELEMS,), jnp.float32),
            pltpu.SemaphoreType.DMA,
        ],
    )(_sc_add_kernel)
    return run(x_flat, enc_flat)


def kernel(x, encoding):
    n, s, d = x.shape
    out_flat = _sc_add(x.reshape(-1), encoding.reshape(-1))
    return out_flat.reshape(n, s, d)
